# baseline (speedup 1.0000x reference)
import jax
import jax.numpy as jnp
from jax.experimental import pallas as pl
from jax.experimental.pallas import tpu as pltpu

N = 524288
D = 128
OUT = 9
COLS = 8192


def _body(x_ref, w_ref, b_ref, o_ref):
    x = x_ref[...]
    yt = jax.lax.dot_general(
        w_ref[...], x, (((0,), (1,)), ((), ())),
        preferred_element_type=jnp.float32,
    )
    ones = jnp.ones((D, 1), dtype=jnp.float32)
    r2t = jax.lax.dot_general(
        ones, x * x, (((0,), (1,)), ((), ())),
        preferred_element_type=jnp.float32,
    )
    ident = (jax.lax.broadcasted_iota(jnp.int32, (OUT, 1), 0) % 4 == 0
             ).astype(jnp.float32)
    scale = jax.lax.rsqrt(r2t)
    y = (yt + b_ref[...] + ident) * scale
    o_ref[...] = jnp.where(r2t < 1e-10, ident, y)


@jax.jit
def kernel(x, W, b):
    yt = pl.pallas_call(
        _body,
        grid=(N // COLS,),
        in_specs=[
            pl.BlockSpec((COLS, D), lambda i: (i, 0)),
            pl.BlockSpec((D, OUT), lambda i: (0, 0)),
            pl.BlockSpec((OUT, 1), lambda i: (0, 0)),
        ],
        out_specs=pl.BlockSpec((OUT, COLS), lambda i: (0, i)),
        out_shape=jax.ShapeDtypeStruct((OUT, N), jnp.float32),
        compiler_params=pltpu.CompilerParams(
            dimension_semantics=("arbitrary",),
        ),
    )(x, W, b.reshape(OUT, 1))
    return yt.T


# speedup vs baseline: 1.1720x; 1.1720x over previous
import jax
import jax.numpy as jnp
from jax.experimental import pallas as pl
from jax.experimental.pallas import tpu as pltpu

N = 524288
D = 128
OUT = 9
COLS = 16384


def _body(x_ref, w_ref, b_ref, o_ref):
    x = x_ref[...]
    yt = jax.lax.dot_general(
        w_ref[...], x, (((0,), (1,)), ((), ())),
        preferred_element_type=jnp.float32,
    )
    ones = jnp.ones((D, 1), dtype=jnp.float32)
    r2t = jax.lax.dot_general(
        ones, x * x, (((0,), (1,)), ((), ())),
        preferred_element_type=jnp.float32,
    )
    ident = (jax.lax.broadcasted_iota(jnp.int32, (OUT, 1), 0) % 4 == 0
             ).astype(jnp.float32)
    scale = jax.lax.rsqrt(r2t)
    y = (yt + b_ref[...] + ident) * scale
    o_ref[...] = jnp.where(r2t < 1e-10, ident, y)


@jax.jit
def kernel(x, W, b):
    yt = pl.pallas_call(
        _body,
        grid=(N // COLS,),
        in_specs=[
            pl.BlockSpec((COLS, D), lambda i: (i, 0)),
            pl.BlockSpec((D, OUT), lambda i: (0, 0)),
            pl.BlockSpec((OUT, 1), lambda i: (0, 0)),
        ],
        out_specs=pl.BlockSpec((OUT, COLS), lambda i: (0, i)),
        out_shape=jax.ShapeDtypeStruct((OUT, N), jnp.float32),
        compiler_params=pltpu.CompilerParams(
            dimension_semantics=("arbitrary",),
        ),
    )(x, W, b.reshape(OUT, 1))
    return yt.T


# speedup vs baseline: 1.2574x; 1.0729x over previous
import jax
import jax.numpy as jnp
from jax.experimental import pallas as pl
from jax.experimental.pallas import tpu as pltpu

N = 524288
D = 128
OUT = 9
COLS = 32768


def _body(x_ref, w_ref, b_ref, o_ref):
    x = x_ref[...]
    yt = jax.lax.dot_general(
        w_ref[...], x, (((0,), (1,)), ((), ())),
        preferred_element_type=jnp.float32,
    )
    ones = jnp.ones((D, 1), dtype=jnp.float32)
    r2t = jax.lax.dot_general(
        ones, x * x, (((0,), (1,)), ((), ())),
        preferred_element_type=jnp.float32,
    )
    ident = (jax.lax.broadcasted_iota(jnp.int32, (OUT, 1), 0) % 4 == 0
             ).astype(jnp.float32)
    scale = jax.lax.rsqrt(r2t)
    y = (yt + b_ref[...] + ident) * scale
    o_ref[...] = jnp.where(r2t < 1e-10, ident, y)


@jax.jit
def kernel(x, W, b):
    yt = pl.pallas_call(
        _body,
        grid=(N // COLS,),
        in_specs=[
            pl.BlockSpec((COLS, D), lambda i: (i, 0)),
            pl.BlockSpec((D, OUT), lambda i: (0, 0)),
            pl.BlockSpec((OUT, 1), lambda i: (0, 0)),
        ],
        out_specs=pl.BlockSpec((OUT, COLS), lambda i: (0, i)),
        out_shape=jax.ShapeDtypeStruct((OUT, N), jnp.float32),
        compiler_params=pltpu.CompilerParams(
            dimension_semantics=("parallel",),
        ),
    )(x, W, b.reshape(OUT, 1))
    return yt.T
